# baseline (speedup 1.0000x reference)
import functools

import jax
import jax.numpy as jnp
from jax import lax
from jax.experimental import pallas as pl
from jax.experimental.pallas import tpu as pltpu
from jax.experimental.pallas import tpu_sc as plsc

N_SRC = 10000
N_DST = 10000
E = 320000
D = 128
ND_PAD = 10240
ROWS_BLK = 2000
GRID = N_SRC // ROWS_BLK

NC = 2
NS = 16
NW = NC * NS
EW = E // NW
K = 80
NCHUNK = EW // K
RPT = ND_PAD // NS
ZR = 64


def _project_body(x_ref, f_ref, wsrc_ref, wdst_ref, asrc_ref, adst_ref,
                  h_ref, als_ref, ald_ref, ms_ref, md_ref):
    i = pl.program_id(0)
    h = jnp.dot(x_ref[...], wsrc_ref[...], preferred_element_type=jnp.float32)
    h_ref[...] = h
    als = jnp.sum(h * asrc_ref[...], axis=1, keepdims=True)
    als_ref[...] = als
    u = jnp.sum(wdst_ref[...] * adst_ref[...], axis=1, keepdims=True)
    ald = jnp.dot(f_ref[...], u, preferred_element_type=jnp.float32)
    ald_ref[...] = ald

    @pl.when(i == 0)
    def _():
        ms_ref[0, 0] = -jnp.inf
        md_ref[0, 0] = -jnp.inf

    ms_ref[0, 0] = jnp.maximum(ms_ref[0, 0], jnp.max(als))
    md_ref[0, 0] = jnp.maximum(md_ref[0, 0], jnp.max(ald))


_project = pl.pallas_call(
    _project_body,
    grid=(GRID,),
    in_specs=[
        pl.BlockSpec((ROWS_BLK, D), lambda i: (i, 0)),
        pl.BlockSpec((ROWS_BLK, D), lambda i: (i, 0)),
        pl.BlockSpec((D, D), lambda i: (0, 0)),
        pl.BlockSpec((D, D), lambda i: (0, 0)),
        pl.BlockSpec((1, D), lambda i: (0, 0)),
        pl.BlockSpec((1, D), lambda i: (0, 0)),
    ],
    out_specs=[
        pl.BlockSpec((ROWS_BLK, D), lambda i: (i, 0)),
        pl.BlockSpec((ROWS_BLK, 1), lambda i: (i, 0)),
        pl.BlockSpec((ROWS_BLK, 1), lambda i: (i, 0)),
        pl.BlockSpec((1, 1), lambda i: (0, 0), memory_space=pltpu.SMEM),
        pl.BlockSpec((1, 1), lambda i: (0, 0), memory_space=pltpu.SMEM),
    ],
    out_shape=[
        jax.ShapeDtypeStruct((N_SRC, D), jnp.float32),
        jax.ShapeDtypeStruct((N_SRC, 1), jnp.float32),
        jax.ShapeDtypeStruct((N_DST, 1), jnp.float32),
        jax.ShapeDtypeStruct((1, 1), jnp.float32),
        jax.ShapeDtypeStruct((1, 1), jnp.float32),
    ],
    compiler_params=pltpu.CompilerParams(
        dimension_semantics=("arbitrary",)),
)


_sc_mesh = plsc.VectorSubcoreMesh(core_axis_name="c", subcore_axis_name="s")


@functools.partial(
    pl.kernel,
    out_type=[
        jax.ShapeDtypeStruct((NC, ND_PAD, D), jnp.float32),
        jax.ShapeDtypeStruct((NC, ND_PAD), jnp.float32),
    ],
    mesh=_sc_mesh,
    compiler_params=pltpu.CompilerParams(needs_layout_passes=False),
    scratch_types=[
        pltpu.VMEM((16,), jnp.float32),
        pltpu.VMEM((K,), jnp.int32),
        pltpu.VMEM((K,), jnp.int32),
        pltpu.VMEM((K,), jnp.int32),
        pltpu.VMEM((K,), jnp.int32),
        pltpu.VMEM((K,), jnp.int32),
        pltpu.VMEM((K,), jnp.int32),
        pltpu.VMEM((K, D), jnp.float32),
        pltpu.VMEM((K, D), jnp.float32),
        pltpu.VMEM((K, D), jnp.float32),
        pltpu.VMEM((K,), jnp.float32),
        pltpu.VMEM((K,), jnp.float32),
        pltpu.VMEM((K,), jnp.float32),
        pltpu.VMEM((K,), jnp.float32),
        pltpu.VMEM((K,), jnp.float32),
        pltpu.VMEM((K,), jnp.float32),
        pltpu.VMEM((K,), jnp.float32),
        pltpu.VMEM((K,), jnp.float32),
        pltpu.VMEM((K,), jnp.float32),
        pltpu.VMEM((K,), jnp.int32),
        pltpu.VMEM((K,), jnp.int32),
        pltpu.VMEM((K,), jnp.int32),
        pltpu.VMEM_SHARED((ND_PAD, D), jnp.float32),
        pltpu.VMEM_SHARED((ND_PAD,), jnp.float32),
        pltpu.VMEM_SHARED((N_SRC,), jnp.float32),
        pltpu.VMEM_SHARED((N_DST,), jnp.float32),
        pltpu.SemaphoreType.DMA,
        pltpu.SemaphoreType.DMA,
        pltpu.SemaphoreType.DMA,
        pltpu.SemaphoreType.DMA,
        pltpu.SemaphoreType.DMA,
        pltpu.SemaphoreType.DMA,
        pltpu.SemaphoreType.DMA,
        pltpu.SemaphoreType.DMA,
        pltpu.SemaphoreType.DMA,
        pltpu.SemaphoreType.DMA,
        pltpu.SemaphoreType.DMA,
        pltpu.SemaphoreType.DMA,
    ],
)
def _sc_edge(eidx_hbm, asrc_hbm, adst_hbm, h_hbm, cvec_hbm, zacc_hbm,
             zden_hbm, acc_out, den_out, cvec_v,
             esv0, esv1, esv2, edv0, edv1, edv2,
             rows0, rows1, rows2, wbuf0, wbuf1, wbuf2, asb0, asb1, asb2,
             adb0, adb1, adb2, eraw0, eraw1, eraw2, acc_sh, den_sh, asrc_sh,
             adst_sh, gsem0, gsem1, gsem2, asem0, asem1, asem2,
             ssem0, ssem1, ssem2, esem0, esem1, esem2):
    cid = lax.axis_index("c")
    sid = lax.axis_index("s")
    wid = sid * NC + cid

    esv = (esv0, esv1, esv2)
    edv = (edv0, edv1, edv2)
    rows = (rows0, rows1, rows2)
    wbuf = (wbuf0, wbuf1, wbuf2)
    asb = (asb0, asb1, asb2)
    adb = (adb0, adb1, adb2)
    eraw = (eraw0, eraw1, eraw2)
    gsem = (gsem0, gsem1, gsem2)
    asem = (asem0, asem1, asem2)
    ssem = (ssem0, ssem1, ssem2)
    esem = (esem0, esem1, esem2)

    pltpu.sync_copy(cvec_hbm, cvec_v)

    @pl.when(sid == 0)
    def _():
        pltpu.sync_copy(asrc_hbm, asrc_sh)
        pltpu.sync_copy(adst_hbm, adst_sh)

    r0 = sid * RPT

    def zloop(i, _):
        pltpu.sync_copy(zacc_hbm, acc_sh.at[pl.ds(r0 + i * ZR, ZR)])
        return 0

    lax.fori_loop(0, RPT // ZR, zloop, 0)
    pltpu.sync_copy(zden_hbm, den_sh.at[pl.ds(r0, RPT)])
    plsc.subcore_barrier()

    cvec = cvec_v[...]
    ebase = wid * EW

    def fetch_idx(c, b):
        pltpu.async_copy(eidx_hbm.at[pl.ds(ebase + c * K, K)], eraw[b],
                         esem[b])

    def launch(c, b):
        pltpu.make_async_copy(eidx_hbm.at[pl.ds(ebase + c * K, K)], eraw[b],
                              esem[b]).wait()

        @plsc.parallel_loop(0, K // 16, 1, unroll=K // 16)
        def _u(g):
            sl = pl.ds(g * 16, 16)
            v = eraw[b][sl]
            esv[b][sl] = lax.shift_right_logical(v, 16)
            edv[b][sl] = lax.bitwise_and(v, 65535)

        pltpu.async_copy(h_hbm.at[esv[b]], rows[b], gsem[b])
        pltpu.async_copy(asrc_sh.at[esv[b]], asb[b], asem[b])
        pltpu.async_copy(adst_sh.at[edv[b]], adb[b], asem[b])

    def finish(c, b):
        pltpu.make_async_copy(asrc_sh.at[esv[b]], asb[b], asem[b]).wait()
        pltpu.make_async_copy(adst_sh.at[edv[b]], adb[b], asem[b]).wait()

        @plsc.parallel_loop(0, K // 16, 1, unroll=K // 16)
        def _w(g):
            raw = asb[b][pl.ds(g * 16, 16)] + adb[b][pl.ds(g * 16, 16)]
            lg = jnp.where(raw >= 0.0, raw, 0.2 * raw)
            wbuf[b][pl.ds(g * 16, 16)] = jnp.exp(lg - cvec)

        pltpu.make_async_copy(h_hbm.at[esv[b]], rows[b], gsem[b]).wait()

        @plsc.parallel_loop(0, K, 1, unroll=4)
        def _s(j):
            jv = jnp.zeros((16,), jnp.int32) + j
            wj = plsc.load_gather(wbuf[b], [jv])
            for r in range(D // 16):
                sl = pl.ds(r * 16, 16)
                rows[b][j, sl] = rows[b][j, sl] * wj

        pltpu.async_copy(rows[b], acc_sh.at[edv[b]], ssem[b], add=True)
        pltpu.async_copy(wbuf[b], den_sh.at[edv[b]], ssem[b], add=True)

    def wait_scatter(b):
        pltpu.make_async_copy(rows[b], acc_sh.at[edv[b]], ssem[b]).wait()
        pltpu.make_async_copy(wbuf[b], den_sh.at[edv[b]], ssem[b]).wait()

    fetch_idx(0, 0)
    fetch_idx(1, 1)
    launch(0, 0)

    def triple(g, _):
        for k in range(3):
            c = 3 * g + k
            b = k
            launch(c + 1, (k + 1) % 3)
            fetch_idx(c + 2, (k + 2) % 3)
            finish(c, b)
            if k == 0:
                @pl.when(g > 0)
                def _():
                    wait_scatter(2)
            else:
                wait_scatter(k - 1)
        return 0

    lax.fori_loop(0, (NCHUNK - 2) // 3, triple, 0)

    launch(NCHUNK - 1, 1)
    finish(NCHUNK - 2, 0)
    wait_scatter(2)
    finish(NCHUNK - 1, 1)
    wait_scatter(0)
    wait_scatter(1)

    plsc.subcore_barrier()
    pltpu.sync_copy(acc_sh.at[pl.ds(r0, RPT)], acc_out.at[cid, pl.ds(r0, RPT)])
    pltpu.sync_copy(den_sh.at[pl.ds(r0, RPT)], den_out.at[cid, pl.ds(r0, RPT)])


def _finalize_body(a0_ref, a1_ref, d0_ref, d1_ref, bias_ref, out_ref):
    s = a0_ref[...].reshape(ROWS_BLK, D) + a1_ref[...].reshape(ROWS_BLK, D)
    den = d0_ref[...].reshape(ROWS_BLK, 1) + d1_ref[...].reshape(ROWS_BLK, 1)
    out_ref[...] = jnp.maximum(s / (den + 1e-16) + bias_ref[...], 0.0)


_finalize = pl.pallas_call(
    _finalize_body,
    grid=(GRID,),
    in_specs=[
        pl.BlockSpec((1, ROWS_BLK, D), lambda i: (0, i, 0)),
        pl.BlockSpec((1, ROWS_BLK, D), lambda i: (1, i, 0)),
        pl.BlockSpec((1, ROWS_BLK, 1), lambda i: (0, i, 0)),
        pl.BlockSpec((1, ROWS_BLK, 1), lambda i: (1, i, 0)),
        pl.BlockSpec((1, D), lambda i: (0, 0)),
    ],
    out_specs=pl.BlockSpec((ROWS_BLK, D), lambda i: (i, 0)),
    out_shape=jax.ShapeDtypeStruct((N_DST, D), jnp.float32),
    compiler_params=pltpu.CompilerParams(
        dimension_semantics=("arbitrary",)),
)


def kernel(pi_edge_index, slice1_feature, slice2_X, W_src, W_dst, a_src,
           a_dst, bias):
    e_src = pi_edge_index[0].astype(jnp.int32)
    e_dst = pi_edge_index[1].astype(jnp.int32)
    eidx = jnp.bitwise_or(jnp.left_shift(e_src, 16), e_dst)

    h_src, als, ald, ms, md = _project(
        slice2_X, slice1_feature, W_src, W_dst,
        a_src.reshape(1, D), a_dst.reshape(1, D))

    c = jnp.maximum(ms[0, 0] + md[0, 0], 0.0)
    cvec = jnp.full((16,), c, jnp.float32)
    zacc = jnp.zeros((ZR, D), jnp.float32)
    zden = jnp.zeros((RPT,), jnp.float32)

    acc, den = _sc_edge(eidx, als.reshape(N_SRC), ald.reshape(N_DST),
                        h_src, cvec, zacc, zden)

    return _finalize(acc, acc, den[:, :, None], den[:, :, None],
                     bias.reshape(1, D))


# speedup vs baseline: 1.1024x; 1.1024x over previous
import functools

import jax
import jax.numpy as jnp
from jax import lax
from jax.experimental import pallas as pl
from jax.experimental.pallas import tpu as pltpu
from jax.experimental.pallas import tpu_sc as plsc

N_SRC = 10000
N_DST = 10000
E = 320000
D = 128
ND_PAD = 10240
ROWS_BLK = 2000
GRID = N_SRC // ROWS_BLK

NC = 2
NS = 16
NW = NC * NS
EW = E // NW
K = 80
NCHUNK = EW // K
RPT = ND_PAD // NS
ZR = 64


def _project_body(x_ref, f_ref, wsrc_ref, wdst_ref, asrc_ref, adst_ref,
                  h_ref, als_ref, ald_ref, cvec_ref, ms_ref, md_ref):
    i = pl.program_id(0)
    h = jnp.dot(x_ref[...], wsrc_ref[...], preferred_element_type=jnp.float32)
    h_ref[...] = h
    als = jnp.sum(h * asrc_ref[...], axis=1, keepdims=True)
    als_ref[...] = als
    u = jnp.sum(wdst_ref[...] * adst_ref[...], axis=1, keepdims=True)
    ald = jnp.dot(f_ref[...], u, preferred_element_type=jnp.float32)
    ald_ref[...] = ald

    @pl.when(i == 0)
    def _():
        ms_ref[0, 0] = -jnp.inf
        md_ref[0, 0] = -jnp.inf

    ms_ref[0, 0] = jnp.maximum(ms_ref[0, 0], jnp.max(als))
    md_ref[0, 0] = jnp.maximum(md_ref[0, 0], jnp.max(ald))

    @pl.when(i == GRID - 1)
    def _():
        c = jnp.maximum(ms_ref[0, 0] + md_ref[0, 0], 0.0)
        cvec_ref[...] = jnp.full((1, 16), c, jnp.float32)


_project = pl.pallas_call(
    _project_body,
    grid=(GRID,),
    in_specs=[
        pl.BlockSpec((ROWS_BLK, D), lambda i: (i, 0)),
        pl.BlockSpec((ROWS_BLK, D), lambda i: (i, 0)),
        pl.BlockSpec((D, D), lambda i: (0, 0)),
        pl.BlockSpec((D, D), lambda i: (0, 0)),
        pl.BlockSpec((1, D), lambda i: (0, 0)),
        pl.BlockSpec((1, D), lambda i: (0, 0)),
    ],
    out_specs=[
        pl.BlockSpec((ROWS_BLK, D), lambda i: (i, 0)),
        pl.BlockSpec((ROWS_BLK, 1), lambda i: (i, 0)),
        pl.BlockSpec((ROWS_BLK, 1), lambda i: (i, 0)),
        pl.BlockSpec((1, 16), lambda i: (0, 0)),
    ],
    out_shape=[
        jax.ShapeDtypeStruct((N_SRC, D), jnp.float32),
        jax.ShapeDtypeStruct((N_SRC, 1), jnp.float32),
        jax.ShapeDtypeStruct((N_DST, 1), jnp.float32),
        jax.ShapeDtypeStruct((1, 16), jnp.float32),
    ],
    scratch_shapes=[
        pltpu.SMEM((1, 1), jnp.float32),
        pltpu.SMEM((1, 1), jnp.float32),
    ],
    compiler_params=pltpu.CompilerParams(
        dimension_semantics=("arbitrary",)),
)


_sc_mesh = plsc.VectorSubcoreMesh(core_axis_name="c", subcore_axis_name="s")


@functools.partial(
    pl.kernel,
    out_type=[
        jax.ShapeDtypeStruct((NC, ND_PAD, D), jnp.float32),
        jax.ShapeDtypeStruct((NC, ND_PAD), jnp.float32),
    ],
    mesh=_sc_mesh,
    compiler_params=pltpu.CompilerParams(needs_layout_passes=False),
    scratch_types=[
        pltpu.VMEM((16,), jnp.float32),
        pltpu.VMEM((K,), jnp.int32),
        pltpu.VMEM((K,), jnp.int32),
        pltpu.VMEM((K,), jnp.int32),
        pltpu.VMEM((K,), jnp.int32),
        pltpu.VMEM((K,), jnp.int32),
        pltpu.VMEM((K,), jnp.int32),
        pltpu.VMEM((K, D), jnp.float32),
        pltpu.VMEM((K, D), jnp.float32),
        pltpu.VMEM((K, D), jnp.float32),
        pltpu.VMEM((K,), jnp.float32),
        pltpu.VMEM((K,), jnp.float32),
        pltpu.VMEM((K,), jnp.float32),
        pltpu.VMEM((K,), jnp.float32),
        pltpu.VMEM((K,), jnp.float32),
        pltpu.VMEM((K,), jnp.float32),
        pltpu.VMEM((K,), jnp.float32),
        pltpu.VMEM((K,), jnp.float32),
        pltpu.VMEM((K,), jnp.float32),
        pltpu.VMEM((K,), jnp.int32),
        pltpu.VMEM((K,), jnp.int32),
        pltpu.VMEM((K,), jnp.int32),
        pltpu.VMEM((ZR, D), jnp.float32),
        pltpu.VMEM((RPT,), jnp.float32),
        pltpu.VMEM_SHARED((ND_PAD, D), jnp.float32),
        pltpu.VMEM_SHARED((ND_PAD,), jnp.float32),
        pltpu.VMEM_SHARED((N_SRC,), jnp.float32),
        pltpu.VMEM_SHARED((N_DST,), jnp.float32),
        pltpu.SemaphoreType.DMA,
        pltpu.SemaphoreType.DMA,
        pltpu.SemaphoreType.DMA,
        pltpu.SemaphoreType.DMA,
        pltpu.SemaphoreType.DMA,
        pltpu.SemaphoreType.DMA,
        pltpu.SemaphoreType.DMA,
        pltpu.SemaphoreType.DMA,
        pltpu.SemaphoreType.DMA,
        pltpu.SemaphoreType.DMA,
        pltpu.SemaphoreType.DMA,
        pltpu.SemaphoreType.DMA,
    ],
)
def _sc_edge(eidx_hbm, asrc_hbm, adst_hbm, h_hbm, cvec_hbm, zacc_hbm,
             zden_hbm, acc_out, den_out, cvec_v,
             esv0, esv1, esv2, edv0, edv1, edv2,
             rows0, rows1, rows2, wbuf0, wbuf1, wbuf2, asb0, asb1, asb2,
             adb0, adb1, adb2, eraw0, eraw1, eraw2, zrow, zden, acc_sh,
             den_sh, asrc_sh,
             adst_sh, gsem0, gsem1, gsem2, asem0, asem1, asem2,
             ssem0, ssem1, ssem2, esem0, esem1, esem2):
    cid = lax.axis_index("c")
    sid = lax.axis_index("s")
    wid = sid * NC + cid

    esv = (esv0, esv1, esv2)
    edv = (edv0, edv1, edv2)
    rows = (rows0, rows1, rows2)
    wbuf = (wbuf0, wbuf1, wbuf2)
    asb = (asb0, asb1, asb2)
    adb = (adb0, adb1, adb2)
    eraw = (eraw0, eraw1, eraw2)
    gsem = (gsem0, gsem1, gsem2)
    asem = (asem0, asem1, asem2)
    ssem = (ssem0, ssem1, ssem2)
    esem = (esem0, esem1, esem2)

    pltpu.sync_copy(cvec_hbm, cvec_v)
    pltpu.sync_copy(zacc_hbm, zrow)
    pltpu.sync_copy(zden_hbm, zden)

    @pl.when(sid == 0)
    def _():
        pltpu.sync_copy(asrc_hbm, asrc_sh)
        pltpu.sync_copy(adst_hbm, adst_sh)

    r0 = sid * RPT

    def zloop(i, _):
        pltpu.sync_copy(zrow, acc_sh.at[pl.ds(r0 + i * ZR, ZR)])
        return 0

    lax.fori_loop(0, RPT // ZR, zloop, 0)
    pltpu.sync_copy(zden, den_sh.at[pl.ds(r0, RPT)])
    plsc.subcore_barrier()

    cvec = cvec_v[...]
    ebase = wid * EW

    def fetch_idx(c, b):
        pltpu.async_copy(eidx_hbm.at[pl.ds(ebase + c * K, K)], eraw[b],
                         esem[b])

    def launch(c, b):
        pltpu.make_async_copy(eidx_hbm.at[pl.ds(ebase + c * K, K)], eraw[b],
                              esem[b]).wait()

        @plsc.parallel_loop(0, K // 16, 1, unroll=K // 16)
        def _u(g):
            sl = pl.ds(g * 16, 16)
            v = eraw[b][sl]
            esv[b][sl] = lax.shift_right_logical(v, 16)
            edv[b][sl] = lax.bitwise_and(v, 65535)

        pltpu.async_copy(h_hbm.at[esv[b]], rows[b], gsem[b])
        pltpu.async_copy(asrc_sh.at[esv[b]], asb[b], asem[b])
        pltpu.async_copy(adst_sh.at[edv[b]], adb[b], asem[b])

    def finish(c, b):
        pltpu.make_async_copy(asrc_sh.at[esv[b]], asb[b], asem[b]).wait()
        pltpu.make_async_copy(adst_sh.at[edv[b]], adb[b], asem[b]).wait()

        @plsc.parallel_loop(0, K // 16, 1, unroll=K // 16)
        def _w(g):
            raw = asb[b][pl.ds(g * 16, 16)] + adb[b][pl.ds(g * 16, 16)]
            lg = jnp.where(raw >= 0.0, raw, 0.2 * raw)
            wbuf[b][pl.ds(g * 16, 16)] = jnp.exp(lg - cvec)

        pltpu.make_async_copy(h_hbm.at[esv[b]], rows[b], gsem[b]).wait()

        @plsc.parallel_loop(0, K, 1, unroll=2)
        def _s(j):
            jv = jnp.zeros((16,), jnp.int32) + j
            wj = plsc.load_gather(wbuf[b], [jv])
            for r in range(D // 16):
                sl = pl.ds(r * 16, 16)
                rows[b][j, sl] = rows[b][j, sl] * wj

        pltpu.async_copy(rows[b], acc_sh.at[edv[b]], ssem[b], add=True)
        pltpu.async_copy(wbuf[b], den_sh.at[edv[b]], ssem[b], add=True)

    def wait_scatter(b):
        pltpu.make_async_copy(rows[b], acc_sh.at[edv[b]], ssem[b]).wait()
        pltpu.make_async_copy(wbuf[b], den_sh.at[edv[b]], ssem[b]).wait()

    fetch_idx(0, 0)
    fetch_idx(1, 1)
    launch(0, 0)

    def triple(g, _):
        for k in range(3):
            c = 3 * g + k
            b = k
            launch(c + 1, (k + 1) % 3)
            fetch_idx(c + 2, (k + 2) % 3)
            finish(c, b)
            if k == 0:
                @pl.when(g > 0)
                def _():
                    wait_scatter(2)
            else:
                wait_scatter(k - 1)
        return 0

    lax.fori_loop(0, (NCHUNK - 2) // 3, triple, 0)

    launch(NCHUNK - 1, 1)
    finish(NCHUNK - 2, 0)
    wait_scatter(2)
    finish(NCHUNK - 1, 1)
    wait_scatter(0)
    wait_scatter(1)

    plsc.subcore_barrier()
    pltpu.sync_copy(acc_sh.at[pl.ds(r0, RPT)], acc_out.at[cid, pl.ds(r0, RPT)])
    pltpu.sync_copy(den_sh.at[pl.ds(r0, RPT)], den_out.at[cid, pl.ds(r0, RPT)])


def _finalize_body(a0_ref, a1_ref, d0_ref, d1_ref, bias_ref, out_ref):
    s = a0_ref[...].reshape(ROWS_BLK, D) + a1_ref[...].reshape(ROWS_BLK, D)
    den = d0_ref[...].reshape(ROWS_BLK, 1) + d1_ref[...].reshape(ROWS_BLK, 1)
    out_ref[...] = jnp.maximum(s / (den + 1e-16) + bias_ref[...], 0.0)


_finalize = pl.pallas_call(
    _finalize_body,
    grid=(GRID,),
    in_specs=[
        pl.BlockSpec((1, ROWS_BLK, D), lambda i: (0, i, 0)),
        pl.BlockSpec((1, ROWS_BLK, D), lambda i: (1, i, 0)),
        pl.BlockSpec((1, ROWS_BLK, 1), lambda i: (0, i, 0)),
        pl.BlockSpec((1, ROWS_BLK, 1), lambda i: (1, i, 0)),
        pl.BlockSpec((1, D), lambda i: (0, 0)),
    ],
    out_specs=pl.BlockSpec((ROWS_BLK, D), lambda i: (i, 0)),
    out_shape=jax.ShapeDtypeStruct((N_DST, D), jnp.float32),
    compiler_params=pltpu.CompilerParams(
        dimension_semantics=("arbitrary",)),
)


def kernel(pi_edge_index, slice1_feature, slice2_X, W_src, W_dst, a_src,
           a_dst, bias):
    e_src = pi_edge_index[0].astype(jnp.int32)
    e_dst = pi_edge_index[1].astype(jnp.int32)
    eidx = jnp.bitwise_or(jnp.left_shift(e_src, 16), e_dst)

    h_src, als, ald, cvec = _project(
        slice2_X, slice1_feature, W_src, W_dst,
        a_src.reshape(1, D), a_dst.reshape(1, D))

    zacc = jnp.zeros((ZR, D), jnp.float32)
    zden = jnp.zeros((RPT,), jnp.float32)

    acc, den = _sc_edge(eidx, als.reshape(N_SRC), ald.reshape(N_DST),
                        h_src, cvec.reshape(16), zacc, zden)

    return _finalize(acc, acc, den[:, :, None], den[:, :, None],
                     bias.reshape(1, D))


# speedup vs baseline: 1.1083x; 1.0053x over previous
import functools

import jax
import jax.numpy as jnp
from jax import lax
from jax.experimental import pallas as pl
from jax.experimental.pallas import tpu as pltpu
from jax.experimental.pallas import tpu_sc as plsc

N_SRC = 10000
N_DST = 10000
E = 320000
D = 128
ND_PAD = 10240
ROWS_BLK = 2000
GRID = N_SRC // ROWS_BLK

NC = 2
NS = 16
NW = NC * NS
EW = E // NW
K = 80
NCHUNK = EW // K
RPT = ND_PAD // NS
ZR = 64


def _project_body(x_ref, f_ref, wsrc_ref, wdst_ref, asrc_ref, adst_ref,
                  h_ref, als_ref, ald_ref, cvec_ref, ms_ref, md_ref):
    i = pl.program_id(0)
    h = jnp.dot(x_ref[...], wsrc_ref[...], preferred_element_type=jnp.float32)
    h_ref[...] = h
    als = jnp.sum(h * asrc_ref[...], axis=1, keepdims=True)
    als_ref[...] = als
    u = jnp.sum(wdst_ref[...] * adst_ref[...], axis=1, keepdims=True)
    ald = jnp.dot(f_ref[...], u, preferred_element_type=jnp.float32)
    ald_ref[...] = ald

    @pl.when(i == 0)
    def _():
        ms_ref[0, 0] = -jnp.inf
        md_ref[0, 0] = -jnp.inf

    ms_ref[0, 0] = jnp.maximum(ms_ref[0, 0], jnp.max(als))
    md_ref[0, 0] = jnp.maximum(md_ref[0, 0], jnp.max(ald))

    @pl.when(i == GRID - 1)
    def _():
        c = jnp.maximum(ms_ref[0, 0] + md_ref[0, 0], 0.0)
        cvec_ref[...] = jnp.full((1, 16), c, jnp.float32)


_project = pl.pallas_call(
    _project_body,
    grid=(GRID,),
    in_specs=[
        pl.BlockSpec((ROWS_BLK, D), lambda i: (i, 0)),
        pl.BlockSpec((ROWS_BLK, D), lambda i: (i, 0)),
        pl.BlockSpec((D, D), lambda i: (0, 0)),
        pl.BlockSpec((D, D), lambda i: (0, 0)),
        pl.BlockSpec((1, D), lambda i: (0, 0)),
        pl.BlockSpec((1, D), lambda i: (0, 0)),
    ],
    out_specs=[
        pl.BlockSpec((ROWS_BLK, D), lambda i: (i, 0)),
        pl.BlockSpec((ROWS_BLK, 1), lambda i: (i, 0)),
        pl.BlockSpec((ROWS_BLK, 1), lambda i: (i, 0)),
        pl.BlockSpec((1, 16), lambda i: (0, 0)),
    ],
    out_shape=[
        jax.ShapeDtypeStruct((N_SRC, D), jnp.float32),
        jax.ShapeDtypeStruct((N_SRC, 1), jnp.float32),
        jax.ShapeDtypeStruct((N_DST, 1), jnp.float32),
        jax.ShapeDtypeStruct((1, 16), jnp.float32),
    ],
    scratch_shapes=[
        pltpu.SMEM((1, 1), jnp.float32),
        pltpu.SMEM((1, 1), jnp.float32),
    ],
    compiler_params=pltpu.CompilerParams(
        dimension_semantics=("arbitrary",)),
)


_sc_mesh = plsc.VectorSubcoreMesh(core_axis_name="c", subcore_axis_name="s")


@functools.partial(
    pl.kernel,
    out_type=[
        jax.ShapeDtypeStruct((NC, ND_PAD, D), jnp.float32),
        jax.ShapeDtypeStruct((NC, ND_PAD), jnp.float32),
    ],
    mesh=_sc_mesh,
    compiler_params=pltpu.CompilerParams(needs_layout_passes=False),
    scratch_types=[
        pltpu.VMEM((16,), jnp.float32),
        pltpu.VMEM((K,), jnp.int32),
        pltpu.VMEM((K,), jnp.int32),
        pltpu.VMEM((K,), jnp.int32),
        pltpu.VMEM((K,), jnp.int32),
        pltpu.VMEM((K,), jnp.int32),
        pltpu.VMEM((K,), jnp.int32),
        pltpu.VMEM((K, D), jnp.float32),
        pltpu.VMEM((K, D), jnp.float32),
        pltpu.VMEM((K, D), jnp.float32),
        pltpu.VMEM((K,), jnp.float32),
        pltpu.VMEM((K,), jnp.float32),
        pltpu.VMEM((K,), jnp.float32),
        pltpu.VMEM((K,), jnp.float32),
        pltpu.VMEM((K,), jnp.float32),
        pltpu.VMEM((K,), jnp.float32),
        pltpu.VMEM((K,), jnp.float32),
        pltpu.VMEM((K,), jnp.float32),
        pltpu.VMEM((K,), jnp.float32),
        pltpu.VMEM((K,), jnp.int32),
        pltpu.VMEM((K,), jnp.int32),
        pltpu.VMEM((K,), jnp.int32),
        pltpu.VMEM((ZR, D), jnp.float32),
        pltpu.VMEM((RPT,), jnp.float32),
        pltpu.VMEM_SHARED((ND_PAD, D), jnp.float32),
        pltpu.VMEM_SHARED((ND_PAD,), jnp.float32),
        pltpu.VMEM_SHARED((N_SRC,), jnp.float32),
        pltpu.VMEM_SHARED((N_DST,), jnp.float32),
        pltpu.SemaphoreType.DMA,
        pltpu.SemaphoreType.DMA,
        pltpu.SemaphoreType.DMA,
        pltpu.SemaphoreType.DMA,
        pltpu.SemaphoreType.DMA,
        pltpu.SemaphoreType.DMA,
        pltpu.SemaphoreType.DMA,
        pltpu.SemaphoreType.DMA,
        pltpu.SemaphoreType.DMA,
        pltpu.SemaphoreType.DMA,
        pltpu.SemaphoreType.DMA,
        pltpu.SemaphoreType.DMA,
        pltpu.SemaphoreType.DMA,
    ],
)
def _sc_edge(eidx_hbm, asrc_hbm, adst_hbm, h_hbm, cvec_hbm, zacc_hbm,
             zden_hbm, acc_out, den_out, cvec_v,
             esv0, esv1, esv2, edv0, edv1, edv2,
             rows0, rows1, rows2, wbuf0, wbuf1, wbuf2, asb0, asb1, asb2,
             adb0, adb1, adb2, eraw0, eraw1, eraw2, zrow, zden, acc_sh,
             den_sh, asrc_sh,
             adst_sh, gsem0, gsem1, gsem2, asem0, asem1, asem2,
             ssem0, ssem1, ssem2, esem0, esem1, esem2, zsem):
    cid = lax.axis_index("c")
    sid = lax.axis_index("s")
    wid = sid * NC + cid

    esv = (esv0, esv1, esv2)
    edv = (edv0, edv1, edv2)
    rows = (rows0, rows1, rows2)
    wbuf = (wbuf0, wbuf1, wbuf2)
    asb = (asb0, asb1, asb2)
    adb = (adb0, adb1, adb2)
    eraw = (eraw0, eraw1, eraw2)
    gsem = (gsem0, gsem1, gsem2)
    asem = (asem0, asem1, asem2)
    ssem = (ssem0, ssem1, ssem2)
    esem = (esem0, esem1, esem2)

    pltpu.sync_copy(cvec_hbm, cvec_v)
    pltpu.sync_copy(zacc_hbm, zrow)
    pltpu.sync_copy(zden_hbm, zden)

    @pl.when(sid == 0)
    def _():
        pltpu.sync_copy(asrc_hbm, asrc_sh)
        pltpu.sync_copy(adst_hbm, adst_sh)

    r0 = sid * RPT

    def zloop(i, _):
        pltpu.async_copy(zrow, acc_sh.at[pl.ds(r0 + i * ZR, ZR)], zsem)
        return 0

    lax.fori_loop(0, RPT // ZR, zloop, 0)
    pltpu.sync_copy(zden, den_sh.at[pl.ds(r0, RPT)])

    def zwait(i, _):
        pltpu.make_async_copy(zrow, acc_sh.at[pl.ds(r0 + i * ZR, ZR)],
                              zsem).wait()
        return 0

    lax.fori_loop(0, RPT // ZR, zwait, 0)
    plsc.subcore_barrier()

    cvec = cvec_v[...]
    ebase = wid * EW

    def fetch_idx(c, b):
        pltpu.async_copy(eidx_hbm.at[pl.ds(ebase + c * K, K)], eraw[b],
                         esem[b])

    def launch(c, b):
        pltpu.make_async_copy(eidx_hbm.at[pl.ds(ebase + c * K, K)], eraw[b],
                              esem[b]).wait()

        @plsc.parallel_loop(0, K // 16, 1, unroll=K // 16)
        def _u(g):
            sl = pl.ds(g * 16, 16)
            v = eraw[b][sl]
            esv[b][sl] = lax.shift_right_logical(v, 16)
            edv[b][sl] = lax.bitwise_and(v, 65535)

        pltpu.async_copy(h_hbm.at[esv[b]], rows[b], gsem[b])
        pltpu.async_copy(asrc_sh.at[esv[b]], asb[b], asem[b])
        pltpu.async_copy(adst_sh.at[edv[b]], adb[b], asem[b])

    def finish(c, b):
        pltpu.make_async_copy(asrc_sh.at[esv[b]], asb[b], asem[b]).wait()
        pltpu.make_async_copy(adst_sh.at[edv[b]], adb[b], asem[b]).wait()

        @plsc.parallel_loop(0, K // 16, 1, unroll=K // 16)
        def _w(g):
            raw = asb[b][pl.ds(g * 16, 16)] + adb[b][pl.ds(g * 16, 16)]
            lg = jnp.where(raw >= 0.0, raw, 0.2 * raw)
            wbuf[b][pl.ds(g * 16, 16)] = jnp.exp(lg - cvec)

        pltpu.make_async_copy(h_hbm.at[esv[b]], rows[b], gsem[b]).wait()

        @plsc.parallel_loop(0, K, 1, unroll=2)
        def _s(j):
            jv = jnp.zeros((16,), jnp.int32) + j
            wj = plsc.load_gather(wbuf[b], [jv])
            for r in range(D // 16):
                sl = pl.ds(r * 16, 16)
                rows[b][j, sl] = rows[b][j, sl] * wj

        pltpu.async_copy(rows[b], acc_sh.at[edv[b]], ssem[b], add=True)
        pltpu.async_copy(wbuf[b], den_sh.at[edv[b]], ssem[b], add=True)

    def wait_scatter(b):
        pltpu.make_async_copy(rows[b], acc_sh.at[edv[b]], ssem[b]).wait()
        pltpu.make_async_copy(wbuf[b], den_sh.at[edv[b]], ssem[b]).wait()

    fetch_idx(0, 0)
    fetch_idx(1, 1)
    launch(0, 0)

    def triple(g, _):
        for k in range(3):
            c = 3 * g + k
            b = k
            launch(c + 1, (k + 1) % 3)
            fetch_idx(c + 2, (k + 2) % 3)
            finish(c, b)
            if k == 0:
                @pl.when(g > 0)
                def _():
                    wait_scatter(2)
            else:
                wait_scatter(k - 1)
        return 0

    lax.fori_loop(0, (NCHUNK - 2) // 3, triple, 0)

    launch(NCHUNK - 1, 1)
    finish(NCHUNK - 2, 0)
    wait_scatter(2)
    finish(NCHUNK - 1, 1)
    wait_scatter(0)
    wait_scatter(1)

    plsc.subcore_barrier()
    cp1 = pltpu.async_copy(acc_sh.at[pl.ds(r0, RPT)],
                           acc_out.at[cid, pl.ds(r0, RPT)], zsem)
    cp2 = pltpu.async_copy(den_sh.at[pl.ds(r0, RPT)],
                           den_out.at[cid, pl.ds(r0, RPT)], zsem)
    cp1.wait()
    cp2.wait()


def _finalize_body(a0_ref, a1_ref, d0_ref, d1_ref, bias_ref, out_ref):
    s = a0_ref[...].reshape(ROWS_BLK, D) + a1_ref[...].reshape(ROWS_BLK, D)
    den = d0_ref[...].reshape(ROWS_BLK, 1) + d1_ref[...].reshape(ROWS_BLK, 1)
    out_ref[...] = jnp.maximum(s / (den + 1e-16) + bias_ref[...], 0.0)


_finalize = pl.pallas_call(
    _finalize_body,
    grid=(GRID,),
    in_specs=[
        pl.BlockSpec((1, ROWS_BLK, D), lambda i: (0, i, 0)),
        pl.BlockSpec((1, ROWS_BLK, D), lambda i: (1, i, 0)),
        pl.BlockSpec((1, ROWS_BLK, 1), lambda i: (0, i, 0)),
        pl.BlockSpec((1, ROWS_BLK, 1), lambda i: (1, i, 0)),
        pl.BlockSpec((1, D), lambda i: (0, 0)),
    ],
    out_specs=pl.BlockSpec((ROWS_BLK, D), lambda i: (i, 0)),
    out_shape=jax.ShapeDtypeStruct((N_DST, D), jnp.float32),
    compiler_params=pltpu.CompilerParams(
        dimension_semantics=("arbitrary",)),
)


def kernel(pi_edge_index, slice1_feature, slice2_X, W_src, W_dst, a_src,
           a_dst, bias):
    e_src = pi_edge_index[0].astype(jnp.int32)
    e_dst = pi_edge_index[1].astype(jnp.int32)
    eidx = jnp.bitwise_or(jnp.left_shift(e_src, 16), e_dst)

    h_src, als, ald, cvec = _project(
        slice2_X, slice1_feature, W_src, W_dst,
        a_src.reshape(1, D), a_dst.reshape(1, D))

    zacc = jnp.zeros((ZR, D), jnp.float32)
    zden = jnp.zeros((RPT,), jnp.float32)

    acc, den = _sc_edge(eidx, als.reshape(N_SRC), ald.reshape(N_DST),
                        h_src, cvec.reshape(16), zacc, zden)

    return _finalize(acc, acc, den[:, :, None], den[:, :, None],
                     bias.reshape(1, D))
